# Initial kernel scaffold; baseline (speedup 1.0000x reference)
import functools

import jax
import jax.numpy as jnp
from jax import lax
from jax.experimental import pallas as pl
from jax.experimental.pallas import tpu as pltpu
from jax.experimental.pallas import tpu_sc as plsc

B = 16384
F = 26
DD = 13
D = 32
OUTF = F + DD

NC, NS = 2, 16
NW = NC * NS
B_PER_W = B // NW
CB = 64
N_CHUNKS = B_PER_W // CB


def _sc_gather(idx_flat, table):
    mesh = plsc.VectorSubcoreMesh(core_axis_name="c", subcore_axis_name="s")

    @functools.partial(
        pl.kernel,
        out_type=jax.ShapeDtypeStruct((B, OUTF, D), jnp.float32),
        mesh=mesh,
        scratch_types=[
            pltpu.VMEM((CB * F,), jnp.int32),
            pltpu.VMEM((CB * F, D), jnp.float32),
            pltpu.SemaphoreType.DMA,
        ],
    )
    def k(idx_hbm, table_hbm, out_hbm, idx_v, rows_v, sem):
        wid = lax.axis_index("s") * NC + lax.axis_index("c")
        base_w = wid * B_PER_W

        @pl.loop(0, N_CHUNKS)
        def _chunk(i):
            base_b = base_w + i * CB
            pltpu.sync_copy(idx_hbm.at[pl.ds(base_b * F, CB * F)], idx_v)
            pltpu.async_copy(table_hbm.at[idx_v], rows_v, sem).wait()
            pltpu.sync_copy(
                rows_v.reshape(CB, F, D),
                out_hbm.at[pl.ds(base_b, CB), pl.ds(0, F), :],
            )

    return k(idx_flat, table)


def _mm_body(x_ref, w_ref, b_ref, _alias_ref, out_ref):
    out_ref[...] = (
        jnp.dot(x_ref[...], w_ref[...], preferred_element_type=jnp.float32)
        + b_ref[...]
    )


def _mm_fill(x, w, b, buf2d):
    BB = 1024
    return pl.pallas_call(
        _mm_body,
        grid=(B // BB,),
        in_specs=[
            pl.BlockSpec((BB, DD), lambda i: (i, 0)),
            pl.BlockSpec((DD, DD * D), lambda i: (0, 0)),
            pl.BlockSpec((1, DD * D), lambda i: (0, 0)),
            pl.BlockSpec(memory_space=pltpu.ANY),
        ],
        out_specs=pl.BlockSpec((BB, DD * D), lambda i: (i, 2)),
        out_shape=jax.ShapeDtypeStruct((B, OUTF * D), jnp.float32),
        input_output_aliases={3: 0},
    )(x, w, b.reshape(1, DD * D), buf2d)


def kernel(inputs_sparse, inputs_dense, emb_table, W, b):
    idx_flat = inputs_sparse.astype(jnp.int32).reshape(-1)
    sc_out = _sc_gather(idx_flat, emb_table)
    out2d = _mm_fill(inputs_dense, W, b, sc_out.reshape(B, OUTF * D))
    return out2d.reshape(B, OUTF, D)


# speedup vs baseline: 1.3853x; 1.3853x over previous
import functools

import jax
import jax.numpy as jnp
from jax import lax
from jax.experimental import pallas as pl
from jax.experimental.pallas import tpu as pltpu
from jax.experimental.pallas import tpu_sc as plsc

B = 16384
F = 26
DD = 13
D = 32
OUTF = F + DD

NC, NS = 2, 16
NW = NC * NS
B_PER_W = B // NW
CB = 64
N_CHUNKS = B_PER_W // CB

_SC_PARAMS = pltpu.CompilerParams(use_tc_tiling_on_sc=False)


def _sc_fill(idx_flat, dst_sp, dense_rows, dst_dn, table):
    mesh = plsc.VectorSubcoreMesh(core_axis_name="c", subcore_axis_name="s")

    @functools.partial(
        pl.kernel,
        out_type=jax.ShapeDtypeStruct((B * OUTF, D), jnp.float32),
        mesh=mesh,
        compiler_params=_SC_PARAMS,
        scratch_types=[
            pltpu.VMEM((CB * F,), jnp.int32),
            pltpu.VMEM((CB * F,), jnp.int32),
            pltpu.VMEM((CB * DD,), jnp.int32),
            pltpu.VMEM((CB * F, D), jnp.float32),
            pltpu.VMEM((CB * DD, D), jnp.float32),
            pltpu.SemaphoreType.DMA,
            pltpu.SemaphoreType.DMA,
        ],
    )
    def k(idx_hbm, dsp_hbm, dense_hbm, ddn_hbm, table_hbm, out_hbm,
          idx_v, dsp_v, ddn_v, rows_v, dense_v, sem_g, sem_s):
        wid = lax.axis_index("s") * NC + lax.axis_index("c")
        base_w = wid * B_PER_W

        @pl.loop(0, N_CHUNKS)
        def _chunk(i):
            base_b = base_w + i * CB
            base_e = base_b * F
            base_d = base_b * DD
            pltpu.sync_copy(idx_hbm.at[pl.ds(base_e, CB * F)], idx_v)
            pltpu.sync_copy(dsp_hbm.at[pl.ds(base_e, CB * F)], dsp_v)
            pltpu.sync_copy(ddn_hbm.at[pl.ds(base_d, CB * DD)], ddn_v)
            gat = pltpu.async_copy(table_hbm.at[idx_v], rows_v, sem_g)
            den = pltpu.async_copy(
                dense_hbm.at[pl.ds(base_d, CB * DD)], dense_v, sem_g)
            gat.wait()
            den.wait()
            sc1 = pltpu.async_copy(rows_v, out_hbm.at[dsp_v], sem_s)
            sc2 = pltpu.async_copy(dense_v, out_hbm.at[ddn_v], sem_s)
            sc1.wait()
            sc2.wait()

    return k(idx_flat, dst_sp, dense_rows, dst_dn, table)


def _mm_body(x_ref, w_ref, b_ref, out_ref):
    out_ref[...] = (
        jnp.dot(x_ref[...], w_ref[...], preferred_element_type=jnp.float32)
        + b_ref[...]
    )


def _mm(x, w, b):
    BB = 2048
    return pl.pallas_call(
        _mm_body,
        grid=(B // BB,),
        in_specs=[
            pl.BlockSpec((BB, DD), lambda i: (i, 0)),
            pl.BlockSpec((DD, DD * D), lambda i: (0, 0)),
            pl.BlockSpec((1, DD * D), lambda i: (0, 0)),
        ],
        out_specs=pl.BlockSpec((BB, DD * D), lambda i: (i, 0)),
        out_shape=jax.ShapeDtypeStruct((B, DD * D), jnp.float32),
    )(x, w, b.reshape(1, DD * D))


def kernel(inputs_sparse, inputs_dense, emb_table, W, b):
    idx_flat = inputs_sparse.astype(jnp.int32).reshape(-1)
    row0 = jnp.arange(B, dtype=jnp.int32) * OUTF
    dst_sp = (row0[:, None] + jnp.arange(F, dtype=jnp.int32)[None, :]).reshape(-1)
    dst_dn = (row0[:, None] + F + jnp.arange(DD, dtype=jnp.int32)[None, :]).reshape(-1)
    dense_rows = _mm(inputs_dense, W, b).reshape(B * DD, D)
    out2d = _sc_fill(idx_flat, dst_sp, dense_rows, dst_dn, emb_table)
    return out2d.reshape(B, OUTF, D)


# speedup vs baseline: 1.9310x; 1.3939x over previous
import functools

import jax
import jax.numpy as jnp
from jax import lax
from jax.experimental import pallas as pl
from jax.experimental.pallas import tpu as pltpu
from jax.experimental.pallas import tpu_sc as plsc

B = 16384
F = 26
DD = 13
D = 32
OUTF = F + DD
N = 1_000_000

NC, NS = 2, 16
NW = NC * NS
B_PER_W = B // NW
CB = 64
N_CHUNKS = B_PER_W // CB

BK = 8192
TGRID = -(-N // BK)
NPAD = TGRID * BK

_SC_PARAMS = pltpu.CompilerParams(use_tc_tiling_on_sc=False)


def _sc_gather(idx_flat, table_rows):
    mesh = plsc.VectorSubcoreMesh(core_axis_name="c", subcore_axis_name="s")

    @functools.partial(
        pl.kernel,
        out_type=jax.ShapeDtypeStruct((B * F, D), jnp.float32),
        mesh=mesh,
        compiler_params=_SC_PARAMS,
        scratch_types=[
            pltpu.VMEM((CB * F,), jnp.int32),
            pltpu.VMEM((CB * F, D), jnp.float32),
            pltpu.SemaphoreType.DMA,
            pltpu.SemaphoreType.DMA,
        ],
    )
    def k(idx_hbm, table_hbm, out_hbm, idx_v, rows_v, sem_g, sem_s):
        wid = lax.axis_index("s") * NC + lax.axis_index("c")
        base_w = wid * B_PER_W

        @pl.loop(0, N_CHUNKS)
        def _chunk(i):
            base_e = (base_w + i * CB) * F
            pltpu.sync_copy(idx_hbm.at[pl.ds(base_e, CB * F)], idx_v)
            gat = pltpu.async_copy(table_hbm.at[idx_v], rows_v, sem_g)
            gat.wait()
            wr = pltpu.async_copy(
                rows_v, out_hbm.at[pl.ds(base_e, CB * F)], sem_s)
            wr.wait()

    return k(idx_flat, table_rows)


CBB = 512
OGRID = B // CBB
GROWS = CBB * F * D // (4 * D)


def _out_body(g_ref, xT_ref, wT_ref, b_ref, out_ref):
    g = g_ref[...]
    dense = (
        jnp.dot(wT_ref[...], xT_ref[...], preferred_element_type=jnp.float32)
        + b_ref[...]
    )
    out_ref[...] = jnp.concatenate([g.T, dense], axis=0)


def _assemble(g2, xT, wT, bcol):
    return pl.pallas_call(
        _out_body,
        grid=(OGRID,),
        in_specs=[
            pl.BlockSpec((CBB, F * D), lambda i: (i, 0)),
            pl.BlockSpec((DD, CBB), lambda i: (0, i)),
            pl.BlockSpec((DD * D, DD), lambda i: (0, 0)),
            pl.BlockSpec((DD * D, 1), lambda i: (0, 0)),
        ],
        out_specs=pl.BlockSpec((OUTF * D, CBB), lambda i: (0, i)),
        out_shape=jax.ShapeDtypeStruct((OUTF * D, B), jnp.float32),
    )(g2, xT, wT, bcol)


def kernel(inputs_sparse, inputs_dense, emb_table, W, b):
    idx_flat = inputs_sparse.astype(jnp.int32).reshape(-1)
    g = _sc_gather(idx_flat, emb_table)
    g2 = g.reshape(B, F * D)
    out2d = _assemble(
        g2,
        inputs_dense.T,
        W.T,
        b.reshape(DD * D, 1),
    )
    return jnp.transpose(out2d.reshape(OUTF, D, B), (2, 0, 1))
